# baseline (speedup 1.0000x reference)
import functools

import jax
import jax.numpy as jnp
from jax import lax
from jax.experimental import pallas as pl
from jax.experimental.pallas import tpu as pltpu
from jax.experimental.pallas import tpu_sc as plsc

LN_EPS = 1e-12
CHUNK = 128


def _sc_gather(ids2d, table):
    n_chunks, _ = ids2d.shape
    _, emb = table.shape
    info = plsc.get_sparse_core_info()
    nc, ns = info.num_cores, info.num_subcores
    nw = nc * ns
    assert n_chunks % nw == 0
    cpw = n_chunks // nw
    n_tok = n_chunks * CHUNK
    ids3d = ids2d.reshape(nw, cpw, CHUNK)

    mesh = plsc.VectorSubcoreMesh(core_axis_name="c", subcore_axis_name="s")

    @functools.partial(
        pl.kernel,
        mesh=mesh,
        out_type=jax.ShapeDtypeStruct((n_tok, emb), table.dtype),
        scratch_types=[
            pltpu.VMEM((cpw, CHUNK), jnp.int32),
            pltpu.VMEM((CHUNK, emb), table.dtype),
            pltpu.VMEM((CHUNK, emb), table.dtype),
            pltpu.SemaphoreType.DMA,
            pltpu.SemaphoreType.DMA,
        ],
    )
    def gather_kernel(ids_hbm, table_hbm, out_hbm, idx_v, buf0, buf1, sem0, sem1):
        wid = lax.axis_index("s") * nc + lax.axis_index("c")
        row0 = wid * cpw * CHUNK
        pltpu.sync_copy(ids_hbm.at[wid], idx_v)

        def gather(j, buf, sem):
            return pltpu.make_async_copy(table_hbm.at[idx_v.at[j]], buf, sem)

        gather(0, buf0, sem0).start()

        def body(i, carry):
            j2 = 2 * i
            gather(j2 + 1, buf1, sem1).start()
            gather(j2, buf0, sem0).wait()
            pltpu.sync_copy(buf0, out_hbm.at[pl.ds(row0 + j2 * CHUNK, CHUNK)])

            @pl.when(j2 + 2 < cpw)
            def _():
                gather(j2 + 2, buf0, sem0).start()

            gather(j2 + 1, buf1, sem1).wait()
            pltpu.sync_copy(buf1, out_hbm.at[pl.ds(row0 + (j2 + 1) * CHUNK, CHUNK)])
            return carry

        lax.fori_loop(0, cpw // 2, body, 0)
        if cpw % 2 == 1:
            gather(cpw - 1, buf0, sem0).wait()
            pltpu.sync_copy(buf0, out_hbm.at[pl.ds(row0 + (cpw - 1) * CHUNK, CHUNK)])

    return gather_kernel(ids3d, table)


def _tc_stage(x3, pos, w, b, total_rows, row_offset, n_rows, prev=None, rows_per_step=16):
    n_tok, emb = x3.shape
    seq = pos.shape[0]
    hid = w.shape[1]
    r = rows_per_step
    assert row_offset % r == 0 and n_rows % r == 0
    assert n_rows * seq <= n_tok
    base = row_offset // r

    def body(*refs):
        x_ref, pos_ref, w_ref, b_ref, o_ref = refs[-5:]
        x = x_ref[...].astype(jnp.float32).reshape(r, seq, emb) + pos_ref[...]
        mean = jnp.mean(x, axis=-1, keepdims=True)
        msq = jnp.mean(x * x, axis=-1, keepdims=True)
        var = msq - mean * mean
        y = (x - mean) * lax.rsqrt(var + LN_EPS)
        res = jnp.dot(
            y.reshape(r * seq, emb), w_ref[...], preferred_element_type=jnp.float32
        )
        o_ref[...] = res.reshape(r, seq, hid) + b_ref[...]

    in_specs = [
        pl.BlockSpec((r * seq, emb), lambda i: (i, 0)),
        pl.BlockSpec((1, seq, emb), lambda i: (0, 0, 0)),
        pl.BlockSpec((emb, hid), lambda i: (0, 0)),
        pl.BlockSpec((1, 1, hid), lambda i: (0, 0, 0)),
    ]
    args = [x3, pos[None], w, b[None]]
    io_alias = {}
    if prev is not None:
        in_specs = [pl.BlockSpec(memory_space=pl.ANY)] + in_specs
        args = [prev] + args
        io_alias = {0: 0}
    return pl.pallas_call(
        body,
        grid=(n_rows // r,),
        in_specs=in_specs,
        out_specs=pl.BlockSpec((r, seq, hid), lambda i: (base + i, 0, 0)),
        out_shape=jax.ShapeDtypeStruct((total_rows, seq, hid), jnp.float32),
        input_output_aliases=io_alias,
    )(*args)


def kernel(input_ids, word_emb, pos_emb, ln_scale, ln_bias, kernel, bias):
    bsz, seq = input_ids.shape
    emb = word_emb.shape[1]
    hid = kernel.shape[1]

    ids2d = input_ids.reshape(-1, CHUNK).astype(jnp.int32)

    w2 = ln_scale[:, None] * kernel
    b2 = (ln_bias @ kernel + bias)[None, :]
    pos = pos_emb[:seq]

    rows1 = 256
    nc1 = rows1 * seq // CHUNK
    nw = 32
    pad1 = (-nc1) % nw
    pad2 = (-(ids2d.shape[0] - nc1)) % nw
    ids_a = jnp.concatenate([ids2d[:nc1], ids2d[:pad1]]) if pad1 else ids2d[:nc1]
    ids_b = (
        jnp.concatenate([ids2d[nc1:], ids2d[:pad2]]) if pad2 else ids2d[nc1:]
    )
    g1 = _sc_gather(ids_a, word_emb)
    g2 = _sc_gather(ids_b, word_emb)
    o1 = _tc_stage(g1, pos, w2, b2, bsz, 0, rows1)
    return _tc_stage(g2, pos, w2, b2, bsz, rows1, bsz - rows1, prev=o1)


# speedup vs baseline: 1.0182x; 1.0182x over previous
import functools

import jax
import jax.numpy as jnp
from jax import lax
from jax.experimental import pallas as pl
from jax.experimental.pallas import tpu as pltpu
from jax.experimental.pallas import tpu_sc as plsc

LN_EPS = 1e-12
CHUNK = 128


def _sc_gather(ids2d, table):
    n_chunks, _ = ids2d.shape
    _, emb = table.shape
    info = plsc.get_sparse_core_info()
    nc, ns = info.num_cores, info.num_subcores
    nw = nc * ns
    assert n_chunks % nw == 0
    cpw = n_chunks // nw
    n_tok = n_chunks * CHUNK
    ids3d = ids2d.reshape(nw, cpw, CHUNK)

    mesh = plsc.VectorSubcoreMesh(core_axis_name="c", subcore_axis_name="s")

    @functools.partial(
        pl.kernel,
        mesh=mesh,
        out_type=jax.ShapeDtypeStruct((n_tok, emb), table.dtype),
        scratch_types=[
            pltpu.VMEM((cpw, CHUNK), jnp.int32),
            pltpu.VMEM((CHUNK, emb), table.dtype),
            pltpu.VMEM((CHUNK, emb), table.dtype),
            pltpu.SemaphoreType.DMA,
            pltpu.SemaphoreType.DMA,
        ],
    )
    def gather_kernel(ids_hbm, table_hbm, out_hbm, idx_v, buf0, buf1, sem0, sem1):
        wid = lax.axis_index("s") * nc + lax.axis_index("c")
        row0 = wid * cpw * CHUNK
        pltpu.sync_copy(ids_hbm.at[wid], idx_v)

        def gather(j, buf, sem):
            return pltpu.make_async_copy(table_hbm.at[idx_v.at[j]], buf, sem)

        gather(0, buf0, sem0).start()

        def body(i, carry):
            j2 = 2 * i
            gather(j2 + 1, buf1, sem1).start()
            gather(j2, buf0, sem0).wait()
            pltpu.sync_copy(buf0, out_hbm.at[pl.ds(row0 + j2 * CHUNK, CHUNK)])

            @pl.when(j2 + 2 < cpw)
            def _():
                gather(j2 + 2, buf0, sem0).start()

            gather(j2 + 1, buf1, sem1).wait()
            pltpu.sync_copy(buf1, out_hbm.at[pl.ds(row0 + (j2 + 1) * CHUNK, CHUNK)])
            return carry

        lax.fori_loop(0, cpw // 2, body, 0)
        if cpw % 2 == 1:
            gather(cpw - 1, buf0, sem0).wait()
            pltpu.sync_copy(buf0, out_hbm.at[pl.ds(row0 + (cpw - 1) * CHUNK, CHUNK)])

    return gather_kernel(ids3d, table)


def _tc_stage(x3, pos, w, b, total_rows, row_offset, n_rows, prev=None, rows_per_step=16):
    n_tok, emb = x3.shape
    seq = pos.shape[0]
    hid = w.shape[1]
    r = rows_per_step
    assert row_offset % r == 0 and n_rows % r == 0
    assert n_rows * seq <= n_tok
    base = row_offset // r

    def body(*refs):
        x_ref, pos_ref, w_ref, b_ref, o_ref = refs[-5:]
        x = x_ref[...].astype(jnp.float32).reshape(r, seq, emb) + pos_ref[...]
        mean = jnp.mean(x, axis=-1, keepdims=True)
        msq = jnp.mean(x * x, axis=-1, keepdims=True)
        var = msq - mean * mean
        y = (x - mean) * lax.rsqrt(var + LN_EPS)
        res = jnp.dot(
            y.reshape(r * seq, emb), w_ref[...], preferred_element_type=jnp.float32
        )
        o_ref[...] = res.reshape(r, seq, hid) + b_ref[...]

    in_specs = [
        pl.BlockSpec((r * seq, emb), lambda i: (i, 0)),
        pl.BlockSpec((1, seq, emb), lambda i: (0, 0, 0)),
        pl.BlockSpec((emb, hid), lambda i: (0, 0)),
        pl.BlockSpec((1, 1, hid), lambda i: (0, 0, 0)),
    ]
    args = [x3, pos[None], w, b[None]]
    io_alias = {}
    if prev is not None:
        in_specs = [pl.BlockSpec(memory_space=pl.ANY)] + in_specs
        args = [prev] + args
        io_alias = {0: 0}
    return pl.pallas_call(
        body,
        grid=(n_rows // r,),
        in_specs=in_specs,
        out_specs=pl.BlockSpec((r, seq, hid), lambda i: (base + i, 0, 0)),
        out_shape=jax.ShapeDtypeStruct((total_rows, seq, hid), jnp.float32),
        input_output_aliases=io_alias,
    )(*args)


def kernel(input_ids, word_emb, pos_emb, ln_scale, ln_bias, kernel, bias):
    bsz, seq = input_ids.shape
    emb = word_emb.shape[1]
    hid = kernel.shape[1]

    ids2d = input_ids.reshape(-1, CHUNK).astype(jnp.int32)

    w2 = ln_scale[:, None] * kernel
    b2 = (ln_bias @ kernel + bias)[None, :]
    pos = pos_emb[:seq]

    g = _sc_gather(ids2d, word_emb)
    return _tc_stage(g, pos, w2, b2, bsz, 0, bsz)
